# baseline (speedup 1.0000x reference)
import functools
import jax
import jax.numpy as jnp
from jax import lax
from jax.experimental import pallas as pl
from jax.experimental.pallas import tpu as pltpu
from jax.experimental.pallas import tpu_sc as plsc

_EMB = 128
_DPAD = 64
_NW = 32
_CHUNK = 128


def _project_body(ents_ref, paths_ref, te_ref, tp_ref, pe_ref, pp_ref):
    pe_ref[...] = jnp.dot(ents_ref[...], te_ref[...],
                          preferred_element_type=jnp.float32
                          ).astype(jnp.bfloat16)
    pp_ref[...] = jnp.dot(paths_ref[...], tp_ref[...],
                          preferred_element_type=jnp.float32
                          ).astype(jnp.bfloat16)


def _project_tables(ents, paths, t_e, t_p):
    vocab = ents.shape[0]
    blk = 2000
    grid = vocab // blk
    out = jax.ShapeDtypeStruct((vocab, 2 * _DPAD), jnp.bfloat16)
    return pl.pallas_call(
        _project_body,
        grid=(grid,),
        in_specs=[
            pl.BlockSpec((blk, _EMB), lambda i: (i, 0)),
            pl.BlockSpec((blk, _EMB), lambda i: (i, 0)),
            pl.BlockSpec((_EMB, 2 * _DPAD), lambda i: (0, 0)),
            pl.BlockSpec((_EMB, 2 * _DPAD), lambda i: (0, 0)),
        ],
        out_specs=[
            pl.BlockSpec((blk, 2 * _DPAD), lambda i: (i, 0)),
            pl.BlockSpec((blk, 2 * _DPAD), lambda i: (i, 0)),
        ],
        out_shape=[out, out],
    )(ents, paths, t_e, t_p)


_GROUPS = 3
_GW = 2
_SLOTS = _GROUPS * _GW
_IW = 128


def _gather_body(n_rows, i1_hbm, i2_hbm, i3_hbm, pe_hbm, pp_hbm,
                 s_hbm, idx1_v, idx2_v, idx3_v, bufs_v, *sems_flat):
    wid = lax.axis_index("s") * 2 + lax.axis_index("c")
    steps = n_rows // _NW
    n_iters = steps // _SLOTS
    w_base = wid * steps

    pltpu.sync_copy(i1_hbm.at[pl.ds(w_base * _IW, steps * _IW)], idx1_v)
    pltpu.sync_copy(i2_hbm.at[pl.ds(w_base * _IW, steps * _IW)], idx2_v)
    pltpu.sync_copy(i3_hbm.at[pl.ds(w_base * _IW, steps * _IW)], idx3_v)

    sems = tuple(sems_flat[3 * g:3 * g + 3] for g in range(_GROUPS))

    def out_desc(d, c, sout):
        return pltpu.make_async_copy(
            bufs_v.at[d],
            s_hbm.at[pl.ds((w_base + c) * _IW, _IW)], sout)

    def run_group(t, g, slots):
        s1, s23, sout = sems[g]
        for d in slots:
            @pl.when(t > 0)
            def _():
                out_desc(d, (t - 1) * _SLOTS + d, sout).wait()
        g1 = []
        for d in slots:
            base = t * _SLOTS + d
            g1.append(pltpu.async_copy(
                pe_hbm.at[idx1_v.at[pl.ds(base * _IW, _IW)]],
                bufs_v.at[d], s1))
        g23 = []
        for i, d in enumerate(slots):
            base = t * _SLOTS + d
            g1[i].wait()
            g23.append(pltpu.async_copy(
                pp_hbm.at[idx2_v.at[pl.ds(base * _IW, _IW)]],
                bufs_v.at[d], s23, add=True))
            g23.append(pltpu.async_copy(
                pe_hbm.at[idx3_v.at[pl.ds(base * _IW, _IW)]],
                bufs_v.at[d], s23, add=True))
        for i, d in enumerate(slots):
            g23[2 * i].wait()
            g23[2 * i + 1].wait()
            out_desc(d, t * _SLOTS + d, sout).start()

    def body(t, _):
        for g in range(_GROUPS):
            run_group(t, g, tuple(range(_GW * g, _GW * g + _GW)))
        return ()

    lax.fori_loop(0, n_iters, body, ())
    for d in range(_SLOTS):
        out_desc(d, (n_iters - 1) * _SLOTS + d, sems[d // _GW][2]).wait()
    for j in range(steps - n_iters * _SLOTS):
        c = n_iters * _SLOTS + j
        pltpu.async_copy(pe_hbm.at[idx1_v.at[pl.ds(c * _IW, _IW)]],
                         bufs_v.at[j], sems[0][0]).wait()
        pltpu.async_copy(pp_hbm.at[idx2_v.at[pl.ds(c * _IW, _IW)]],
                         bufs_v.at[j], sems[0][1], add=True).wait()
        pltpu.async_copy(pe_hbm.at[idx3_v.at[pl.ds(c * _IW, _IW)]],
                         bufs_v.at[j], sems[0][1], add=True).wait()
        od = out_desc(j, c, sems[0][2])
        od.start()
        od.wait()


def _gather_sum(i1, i2, i3, pe, pp):
    n_rows = i1.shape[0] // _IW
    rows_w = n_rows // _NW
    mesh = plsc.VectorSubcoreMesh(core_axis_name="c", subcore_axis_name="s")
    kern = pl.kernel(
        functools.partial(_gather_body, n_rows),
        out_type=jax.ShapeDtypeStruct((n_rows * _IW, _DPAD), jnp.bfloat16),
        mesh=mesh,
        scratch_types=[
            pltpu.VMEM((rows_w * _IW,), jnp.int32),
            pltpu.VMEM((rows_w * _IW,), jnp.int32),
            pltpu.VMEM((rows_w * _IW,), jnp.int32),
            pltpu.VMEM((_SLOTS, _IW, _DPAD), jnp.bfloat16),
        ] + [pltpu.SemaphoreType.DMA] * (3 * _GROUPS),
        compiler_params=pltpu.CompilerParams(use_tc_tiling_on_sc=False),
    )
    return kern(i1, i2, i3, pe, pp)


def _attend_body(blk, half, s_ref, a_ref, ae_ref, ao_ref, cv_ref):
    t = jnp.tanh(s_ref[...].astype(jnp.float32))
    w = jnp.dot(t, a_ref[...], preferred_element_type=jnp.float32)
    w3 = w.reshape(blk, half, 2 * _DPAD)
    m1 = jnp.max(w3, axis=1)
    m = jnp.maximum(m1[:, :_DPAD], m1[:, _DPAD:])
    mfull = jnp.concatenate([m, m], axis=-1)
    e3 = jnp.exp(w3 - mfull[:, None, :])
    d2 = jnp.sum(e3, axis=1)
    d = d2[:, :_DPAD] + d2[:, _DPAD:]
    rdfull = 1.0 / jnp.concatenate([d, d], axis=-1)
    attn3 = e3 * rdfull[:, None, :]
    y = jnp.sum(t.reshape(blk, half, 2 * _DPAD) * attn3, axis=1)
    cv_ref[...] = y[:, :_DPAD] + y[:, _DPAD:]
    ae_ref[...] = jnp.max(attn3[:, :, :_DPAD], axis=-1)
    ao_ref[...] = jnp.max(attn3[:, :, _DPAD:], axis=-1)


def _attend(s2, a_blk, batch, ctx):
    blk = 128
    half = ctx // 2
    grid = batch // blk
    rows = blk * half
    return pl.pallas_call(
        functools.partial(_attend_body, blk, half),
        grid=(grid,),
        in_specs=[
            pl.BlockSpec((rows, 2 * _DPAD), lambda i: (i, 0)),
            pl.BlockSpec((2 * _DPAD, 2 * _DPAD), lambda i: (0, 0)),
        ],
        out_specs=[
            pl.BlockSpec((blk, half), lambda i: (i, 0)),
            pl.BlockSpec((blk, half), lambda i: (i, 0)),
            pl.BlockSpec((blk, _DPAD), lambda i: (i, 0)),
        ],
        out_shape=[
            jax.ShapeDtypeStruct((batch, half), jnp.float32),
            jax.ShapeDtypeStruct((batch, half), jnp.float32),
            jax.ShapeDtypeStruct((batch, _DPAD), jnp.float32),
        ],
    )(s2, a_blk)


def kernel(e1, p, e2, ents_embeddings, path_embeddings, transform_matrix,
           attention_param):
    batch, ctx = e1.shape
    code = transform_matrix.shape[1]

    t3 = transform_matrix.reshape(3, _EMB, code)
    t_pad = jnp.zeros((3, _EMB, _DPAD), jnp.float32).at[:, :, :code].set(t3)
    t_e = jnp.concatenate([t_pad[0], t_pad[2]], axis=1)
    t_p = jnp.concatenate([t_pad[1], t_pad[1]], axis=1)
    a64 = jnp.zeros((_DPAD,), jnp.float32).at[:code].set(attention_param[:, 0])
    a_rep = jnp.outer(a64, jnp.ones((_DPAD,), jnp.float32))
    z64 = jnp.zeros((_DPAD, _DPAD), jnp.float32)
    a_blk = jnp.block([[a_rep, z64], [z64, a_rep]])

    pe, pp = _project_tables(ents_embeddings, path_embeddings, t_e, t_p)
    vocab2 = 2 * pe.shape[0]
    pe_v = pe.reshape(vocab2, _DPAD)
    pp_v = pp.reshape(vocab2, _DPAD)

    nsplit = 4
    bsub = batch // nsplit
    nsub = bsub * ctx
    outs = []
    for k in range(nsplit):
        lo = k * bsub
        i1 = (e1[lo:lo + bsub].reshape(-1) * 2).astype(jnp.int32)
        i2 = (p[lo:lo + bsub].reshape(-1) * 2).astype(jnp.int32)
        i3 = (e2[lo:lo + bsub].reshape(-1) * 2 + 1).astype(jnp.int32)
        s = _gather_sum(i1, i2, i3, pe_v, pp_v)
        outs.append(_attend(s.reshape(nsub // 2, 2 * _DPAD), a_blk,
                            bsub, ctx))

    ae = jnp.concatenate([o[0] for o in outs])
    ao = jnp.concatenate([o[1] for o in outs])
    cv = jnp.concatenate([o[2] for o in outs])
    attn = jnp.stack([ae, ao], axis=-1).reshape(batch, ctx, 1)
    return cv[:, :code], attn


# speedup vs baseline: 1.8336x; 1.8336x over previous
import functools
import jax
import jax.numpy as jnp
from jax import lax
from jax.experimental import pallas as pl
from jax.experimental.pallas import tpu as pltpu
from jax.experimental.pallas import tpu_sc as plsc

_EMB = 128
_DPAD = 64
_NW = 32
_CHUNK = 128


def _project_body(ents_ref, paths_ref, te_ref, tp_ref, pe_ref, pp_ref):
    pe_ref[...] = jnp.dot(ents_ref[...], te_ref[...],
                          preferred_element_type=jnp.float32)
    pp_ref[...] = jnp.dot(paths_ref[...], tp_ref[...],
                          preferred_element_type=jnp.float32)


def _project_tables(ents, paths, t_e, t_p):
    vocab = ents.shape[0]
    blk = 2000
    grid = vocab // blk
    out = jax.ShapeDtypeStruct((vocab, 2 * _DPAD), jnp.float32)
    return pl.pallas_call(
        _project_body,
        grid=(grid,),
        in_specs=[
            pl.BlockSpec((blk, _EMB), lambda i: (i, 0)),
            pl.BlockSpec((blk, _EMB), lambda i: (i, 0)),
            pl.BlockSpec((_EMB, 2 * _DPAD), lambda i: (0, 0)),
            pl.BlockSpec((_EMB, 2 * _DPAD), lambda i: (0, 0)),
        ],
        out_specs=[
            pl.BlockSpec((blk, 2 * _DPAD), lambda i: (i, 0)),
            pl.BlockSpec((blk, 2 * _DPAD), lambda i: (i, 0)),
        ],
        out_shape=[out, out],
    )(ents, paths, t_e, t_p)


_GROUPS = 3
_GW = 2
_SLOTS = _GROUPS * _GW
_IW = 128


def _gather_body(n_rows, i1_hbm, i2_hbm, i3_hbm, pe_hbm, pp_hbm,
                 s_hbm, idx1_v, idx2_v, idx3_v, bufs_v, *sems_flat):
    wid = lax.axis_index("s") * 2 + lax.axis_index("c")
    steps = n_rows // _NW
    n_iters = steps // _SLOTS
    w_base = wid * steps

    pltpu.sync_copy(i1_hbm.at[pl.ds(w_base * _IW, steps * _IW)], idx1_v)
    pltpu.sync_copy(i2_hbm.at[pl.ds(w_base * _IW, steps * _IW)], idx2_v)
    pltpu.sync_copy(i3_hbm.at[pl.ds(w_base * _IW, steps * _IW)], idx3_v)

    sems = tuple(sems_flat[3 * g:3 * g + 3] for g in range(_GROUPS))

    def out_desc(d, c, sout):
        return pltpu.make_async_copy(
            bufs_v.at[d],
            s_hbm.at[pl.ds((w_base + c) * _IW, _IW)], sout)

    def run_group(t, g, slots):
        s1, s23, sout = sems[g]
        for d in slots:
            @pl.when(t > 0)
            def _():
                out_desc(d, (t - 1) * _SLOTS + d, sout).wait()
        g1 = []
        for d in slots:
            base = t * _SLOTS + d
            g1.append(pltpu.async_copy(
                pe_hbm.at[idx1_v.at[pl.ds(base * _IW, _IW)]],
                bufs_v.at[d], s1))
        g23 = []
        for i, d in enumerate(slots):
            base = t * _SLOTS + d
            g1[i].wait()
            g23.append(pltpu.async_copy(
                pp_hbm.at[idx2_v.at[pl.ds(base * _IW, _IW)]],
                bufs_v.at[d], s23, add=True))
            g23.append(pltpu.async_copy(
                pe_hbm.at[idx3_v.at[pl.ds(base * _IW, _IW)]],
                bufs_v.at[d], s23, add=True))
        for i, d in enumerate(slots):
            g23[2 * i].wait()
            g23[2 * i + 1].wait()
            out_desc(d, t * _SLOTS + d, sout).start()

    def body(t, _):
        for g in range(_GROUPS):
            run_group(t, g, tuple(range(_GW * g, _GW * g + _GW)))
        return ()

    lax.fori_loop(0, n_iters, body, ())
    for d in range(_SLOTS):
        out_desc(d, (n_iters - 1) * _SLOTS + d, sems[d // _GW][2]).wait()
    for j in range(steps - n_iters * _SLOTS):
        c = n_iters * _SLOTS + j
        pltpu.async_copy(pe_hbm.at[idx1_v.at[pl.ds(c * _IW, _IW)]],
                         bufs_v.at[j], sems[0][0]).wait()
        pltpu.async_copy(pp_hbm.at[idx2_v.at[pl.ds(c * _IW, _IW)]],
                         bufs_v.at[j], sems[0][1], add=True).wait()
        pltpu.async_copy(pe_hbm.at[idx3_v.at[pl.ds(c * _IW, _IW)]],
                         bufs_v.at[j], sems[0][1], add=True).wait()
        od = out_desc(j, c, sems[0][2])
        od.start()
        od.wait()


def _gather_sum(i1, i2, i3, pe, pp):
    n_rows = i1.shape[0] // _IW
    rows_w = n_rows // _NW
    mesh = plsc.VectorSubcoreMesh(core_axis_name="c", subcore_axis_name="s")
    kern = pl.kernel(
        functools.partial(_gather_body, n_rows),
        out_type=jax.ShapeDtypeStruct((n_rows * _IW, _DPAD), jnp.float32),
        mesh=mesh,
        scratch_types=[
            pltpu.VMEM((rows_w * _IW,), jnp.int32),
            pltpu.VMEM((rows_w * _IW,), jnp.int32),
            pltpu.VMEM((rows_w * _IW,), jnp.int32),
            pltpu.VMEM((_SLOTS, _IW, _DPAD), jnp.float32),
        ] + [pltpu.SemaphoreType.DMA] * (3 * _GROUPS),
        compiler_params=pltpu.CompilerParams(use_tc_tiling_on_sc=False),
    )
    return kern(i1, i2, i3, pe, pp)


def _attend_body(blk, half, s_ref, a_ref, ae_ref, ao_ref, cv_ref):
    t = jnp.tanh(s_ref[...])
    w = jnp.dot(t, a_ref[...], preferred_element_type=jnp.float32)
    w3 = w.reshape(blk, half, 2 * _DPAD)
    m1 = jnp.max(w3, axis=1)
    m = jnp.maximum(m1[:, :_DPAD], m1[:, _DPAD:])
    mfull = jnp.concatenate([m, m], axis=-1)
    e3 = jnp.exp(w3 - mfull[:, None, :])
    d2 = jnp.sum(e3, axis=1)
    d = d2[:, :_DPAD] + d2[:, _DPAD:]
    rdfull = 1.0 / jnp.concatenate([d, d], axis=-1)
    attn3 = e3 * rdfull[:, None, :]
    y = jnp.sum(t.reshape(blk, half, 2 * _DPAD) * attn3, axis=1)
    cv_ref[...] = y[:, :_DPAD] + y[:, _DPAD:]
    ae_ref[...] = jnp.max(attn3[:, :, :_DPAD], axis=-1)
    ao_ref[...] = jnp.max(attn3[:, :, _DPAD:], axis=-1)


def _attend(s2, a_blk, batch, ctx):
    blk = 128
    half = ctx // 2
    grid = batch // blk
    rows = blk * half
    return pl.pallas_call(
        functools.partial(_attend_body, blk, half),
        grid=(grid,),
        in_specs=[
            pl.BlockSpec((rows, 2 * _DPAD), lambda i: (i, 0)),
            pl.BlockSpec((2 * _DPAD, 2 * _DPAD), lambda i: (0, 0)),
        ],
        out_specs=[
            pl.BlockSpec((blk, half), lambda i: (i, 0)),
            pl.BlockSpec((blk, half), lambda i: (i, 0)),
            pl.BlockSpec((blk, _DPAD), lambda i: (i, 0)),
        ],
        out_shape=[
            jax.ShapeDtypeStruct((batch, half), jnp.float32),
            jax.ShapeDtypeStruct((batch, half), jnp.float32),
            jax.ShapeDtypeStruct((batch, _DPAD), jnp.float32),
        ],
    )(s2, a_blk)


def kernel(e1, p, e2, ents_embeddings, path_embeddings, transform_matrix,
           attention_param):
    batch, ctx = e1.shape
    code = transform_matrix.shape[1]

    t3 = transform_matrix.reshape(3, _EMB, code)
    t_pad = jnp.zeros((3, _EMB, _DPAD), jnp.float32).at[:, :, :code].set(t3)
    t_e = jnp.concatenate([t_pad[0], t_pad[2]], axis=1)
    t_p = jnp.concatenate([t_pad[1], t_pad[1]], axis=1)
    a64 = jnp.zeros((_DPAD,), jnp.float32).at[:code].set(attention_param[:, 0])
    a_rep = jnp.outer(a64, jnp.ones((_DPAD,), jnp.float32))
    z64 = jnp.zeros((_DPAD, _DPAD), jnp.float32)
    a_blk = jnp.block([[a_rep, z64], [z64, a_rep]])

    pe, pp = _project_tables(ents_embeddings, path_embeddings, t_e, t_p)
    vocab2 = 2 * pe.shape[0]
    pe_v = pe.reshape(vocab2, _DPAD)
    pp_v = pp.reshape(vocab2, _DPAD)

    nsplit = 4
    bsub = batch // nsplit
    nsub = bsub * ctx
    outs = []
    for k in range(nsplit):
        lo = k * bsub
        i1 = (e1[lo:lo + bsub].reshape(-1) * 2).astype(jnp.int32)
        i2 = (p[lo:lo + bsub].reshape(-1) * 2).astype(jnp.int32)
        i3 = (e2[lo:lo + bsub].reshape(-1) * 2 + 1).astype(jnp.int32)
        s = _gather_sum(i1, i2, i3, pe_v, pp_v)
        outs.append(_attend(s.reshape(nsub // 2, 2 * _DPAD), a_blk,
                            bsub, ctx))

    ae = jnp.concatenate([o[0] for o in outs])
    ao = jnp.concatenate([o[1] for o in outs])
    cv = jnp.concatenate([o[2] for o in outs])
    attn = jnp.stack([ae, ao], axis=-1).reshape(batch, ctx, 1)
    return cv[:, :code], attn


# speedup vs baseline: 1.8926x; 1.0322x over previous
import functools
import jax
import jax.numpy as jnp
from jax import lax
from jax.experimental import pallas as pl
from jax.experimental.pallas import tpu as pltpu
from jax.experimental.pallas import tpu_sc as plsc

_EMB = 128
_DPAD = 64
_NW = 32
_CHUNK = 128


def _project_body(ents_ref, paths_ref, te_ref, tp_ref, pe_ref, pp_ref):
    pe_ref[...] = jnp.dot(ents_ref[...], te_ref[...],
                          preferred_element_type=jnp.float32)
    pp_ref[...] = jnp.dot(paths_ref[...], tp_ref[...],
                          preferred_element_type=jnp.float32)


def _project_tables(ents, paths, t_e, t_p):
    vocab = ents.shape[0]
    blk = 2000
    grid = vocab // blk
    out = jax.ShapeDtypeStruct((vocab, 2 * _DPAD), jnp.float32)
    return pl.pallas_call(
        _project_body,
        grid=(grid,),
        in_specs=[
            pl.BlockSpec((blk, _EMB), lambda i: (i, 0)),
            pl.BlockSpec((blk, _EMB), lambda i: (i, 0)),
            pl.BlockSpec((_EMB, 2 * _DPAD), lambda i: (0, 0)),
            pl.BlockSpec((_EMB, 2 * _DPAD), lambda i: (0, 0)),
        ],
        out_specs=[
            pl.BlockSpec((blk, 2 * _DPAD), lambda i: (i, 0)),
            pl.BlockSpec((blk, 2 * _DPAD), lambda i: (i, 0)),
        ],
        out_shape=[out, out],
    )(ents, paths, t_e, t_p)


_GROUPS = 3
_GW = 2
_SLOTS = _GROUPS * _GW
_IW = 128


def _gather_body(n_rows, i1_hbm, i2_hbm, i3_hbm, pe_hbm, pp_hbm,
                 s_hbm, idx1_v, idx2_v, idx3_v, bufs_v, *sems_flat):
    wid = lax.axis_index("s") * 2 + lax.axis_index("c")
    steps = n_rows // _NW
    n_iters = steps // _SLOTS
    w_base = wid * steps

    pltpu.sync_copy(i1_hbm.at[pl.ds(w_base * _IW, steps * _IW)], idx1_v)
    pltpu.sync_copy(i2_hbm.at[pl.ds(w_base * _IW, steps * _IW)], idx2_v)
    pltpu.sync_copy(i3_hbm.at[pl.ds(w_base * _IW, steps * _IW)], idx3_v)

    sems = tuple(sems_flat[3 * g:3 * g + 3] for g in range(_GROUPS))

    def out_desc(d, c, sout):
        return pltpu.make_async_copy(
            bufs_v.at[d],
            s_hbm.at[pl.ds((w_base + c) * _IW, _IW)], sout)

    def run_group(t, g, slots):
        s1, s23, sout = sems[g]
        for d in slots:
            @pl.when(t > 0)
            def _():
                out_desc(d, (t - 1) * _SLOTS + d, sout).wait()
        g1 = []
        for d in slots:
            base = t * _SLOTS + d
            g1.append(pltpu.async_copy(
                pe_hbm.at[idx1_v.at[pl.ds(base * _IW, _IW)]],
                bufs_v.at[d], s1))
        g23 = []
        for i, d in enumerate(slots):
            base = t * _SLOTS + d
            g1[i].wait()
            g23.append(pltpu.async_copy(
                pp_hbm.at[idx2_v.at[pl.ds(base * _IW, _IW)]],
                bufs_v.at[d], s23, add=True))
            g23.append(pltpu.async_copy(
                pe_hbm.at[idx3_v.at[pl.ds(base * _IW, _IW)]],
                bufs_v.at[d], s23, add=True))
        for i, d in enumerate(slots):
            g23[2 * i].wait()
            g23[2 * i + 1].wait()
            out_desc(d, t * _SLOTS + d, sout).start()

    def body(t, _):
        for g in range(_GROUPS):
            run_group(t, g, tuple(range(_GW * g, _GW * g + _GW)))
        return ()

    lax.fori_loop(0, n_iters, body, ())
    for d in range(_SLOTS):
        out_desc(d, (n_iters - 1) * _SLOTS + d, sems[d // _GW][2]).wait()
    for j in range(steps - n_iters * _SLOTS):
        c = n_iters * _SLOTS + j
        pltpu.async_copy(pe_hbm.at[idx1_v.at[pl.ds(c * _IW, _IW)]],
                         bufs_v.at[j], sems[0][0]).wait()
        pltpu.async_copy(pp_hbm.at[idx2_v.at[pl.ds(c * _IW, _IW)]],
                         bufs_v.at[j], sems[0][1], add=True).wait()
        pltpu.async_copy(pe_hbm.at[idx3_v.at[pl.ds(c * _IW, _IW)]],
                         bufs_v.at[j], sems[0][1], add=True).wait()
        od = out_desc(j, c, sems[0][2])
        od.start()
        od.wait()


def _gather_sum(i1, i2, i3, pe, pp):
    n_rows = i1.shape[0] // _IW
    rows_w = n_rows // _NW
    mesh = plsc.VectorSubcoreMesh(core_axis_name="c", subcore_axis_name="s")
    kern = pl.kernel(
        functools.partial(_gather_body, n_rows),
        out_type=jax.ShapeDtypeStruct((n_rows * _IW, _DPAD), jnp.float32),
        mesh=mesh,
        scratch_types=[
            pltpu.VMEM((rows_w * _IW,), jnp.int32),
            pltpu.VMEM((rows_w * _IW,), jnp.int32),
            pltpu.VMEM((rows_w * _IW,), jnp.int32),
            pltpu.VMEM((_SLOTS, _IW, _DPAD), jnp.float32),
        ] + [pltpu.SemaphoreType.DMA] * (3 * _GROUPS),
        compiler_params=pltpu.CompilerParams(use_tc_tiling_on_sc=False),
    )
    return kern(i1, i2, i3, pe, pp)


def _attend_body(blk, half, s_ref, a_ref, ae_ref, ao_ref, cv_ref):
    t = jnp.tanh(s_ref[...])
    w = jnp.dot(t, a_ref[...], preferred_element_type=jnp.float32)
    w3 = w.reshape(blk, half, 2 * _DPAD)
    m1 = jnp.max(w3, axis=1)
    m = jnp.maximum(m1[:, :_DPAD], m1[:, _DPAD:])
    mfull = jnp.concatenate([m, m], axis=-1)
    e3 = jnp.exp(w3 - mfull[:, None, :])
    d2 = jnp.sum(e3, axis=1)
    d = d2[:, :_DPAD] + d2[:, _DPAD:]
    rdfull = 1.0 / jnp.concatenate([d, d], axis=-1)
    attn3 = e3 * rdfull[:, None, :]
    y = jnp.sum(t.reshape(blk, half, 2 * _DPAD) * attn3, axis=1)
    cv_ref[...] = y[:, :_DPAD] + y[:, _DPAD:]
    ae_ref[...] = jnp.max(attn3[:, :, :_DPAD], axis=-1)
    ao_ref[...] = jnp.max(attn3[:, :, _DPAD:], axis=-1)


def _attend(s2, a_blk, batch, ctx):
    blk = 128
    half = ctx // 2
    grid = batch // blk
    rows = blk * half
    return pl.pallas_call(
        functools.partial(_attend_body, blk, half),
        grid=(grid,),
        in_specs=[
            pl.BlockSpec((rows, 2 * _DPAD), lambda i: (i, 0)),
            pl.BlockSpec((2 * _DPAD, 2 * _DPAD), lambda i: (0, 0)),
        ],
        out_specs=[
            pl.BlockSpec((blk, half), lambda i: (i, 0)),
            pl.BlockSpec((blk, half), lambda i: (i, 0)),
            pl.BlockSpec((blk, _DPAD), lambda i: (i, 0)),
        ],
        out_shape=[
            jax.ShapeDtypeStruct((batch, half), jnp.float32),
            jax.ShapeDtypeStruct((batch, half), jnp.float32),
            jax.ShapeDtypeStruct((batch, _DPAD), jnp.float32),
        ],
    )(s2, a_blk)


def kernel(e1, p, e2, ents_embeddings, path_embeddings, transform_matrix,
           attention_param):
    batch, ctx = e1.shape
    code = transform_matrix.shape[1]

    t3 = transform_matrix.reshape(3, _EMB, code)
    t_pad = jnp.zeros((3, _EMB, _DPAD), jnp.float32).at[:, :, :code].set(t3)
    t_e = jnp.concatenate([t_pad[0], t_pad[2]], axis=1)
    t_p = jnp.concatenate([t_pad[1], t_pad[1]], axis=1)
    a64 = jnp.zeros((_DPAD,), jnp.float32).at[:code].set(attention_param[:, 0])
    a_rep = jnp.outer(a64, jnp.ones((_DPAD,), jnp.float32))
    z64 = jnp.zeros((_DPAD, _DPAD), jnp.float32)
    a_blk = jnp.block([[a_rep, z64], [z64, a_rep]])

    pe, pp = _project_tables(ents_embeddings, path_embeddings, t_e, t_p)
    vocab2 = 2 * pe.shape[0]
    pe_v = pe.reshape(vocab2, _DPAD)
    pp_v = pp.reshape(vocab2, _DPAD)

    splits = [1024, 1024, 1024, 512, 512]
    outs = []
    lo = 0
    for bsub in splits:
        nsub = bsub * ctx
        i1 = (e1[lo:lo + bsub].reshape(-1) * 2).astype(jnp.int32)
        i2 = (p[lo:lo + bsub].reshape(-1) * 2).astype(jnp.int32)
        i3 = (e2[lo:lo + bsub].reshape(-1) * 2 + 1).astype(jnp.int32)
        s = _gather_sum(i1, i2, i3, pe_v, pp_v)
        outs.append(_attend(s.reshape(nsub // 2, 2 * _DPAD), a_blk,
                            bsub, ctx))
        lo += bsub

    ae = jnp.concatenate([o[0] for o in outs])
    ao = jnp.concatenate([o[1] for o in outs])
    cv = jnp.concatenate([o[2] for o in outs])
    attn = jnp.stack([ae, ao], axis=-1).reshape(batch, ctx, 1)
    return cv[:, :code], attn
